# Initial kernel scaffold; baseline (speedup 1.0000x reference)
import jax
import jax.numpy as jnp
from jax.experimental import pallas as pl


def kernel(token_ids, weight):
    raise NotImplementedError("write your pallas kernel here")


# speedup vs baseline: 1.8410x; 1.8410x over previous
import functools

import jax
import jax.numpy as jnp
from jax import lax
from jax.experimental import pallas as pl
from jax.experimental.pallas import tpu as pltpu
from jax.experimental.pallas import tpu_sc as plsc

_NC = 2
_NS = 16
_NW = _NC * _NS
_D = 64
_GRP = 128
_G = 5
_CHUNK = _G * _GRP


@functools.cache
def _make_lookup(b_per_w: int):
    n_grp = b_per_w // _GRP
    n_chunks = b_per_w // _CHUNK
    mesh = plsc.VectorSubcoreMesh(core_axis_name="c", subcore_axis_name="s")

    @functools.partial(
        pl.kernel,
        mesh=mesh,
        compiler_params=pltpu.CompilerParams(use_tc_tiling_on_sc=False),
        out_type=jax.ShapeDtypeStruct((_NW, b_per_w, _D), jnp.float32),
        scratch_types=[
            pltpu.VMEM((n_grp, _GRP), jnp.int32),
            pltpu.VMEM((_CHUNK, _D), jnp.float32),
            pltpu.SemaphoreType.DMA,
        ],
    )
    def lookup(table_hbm, idx_hbm, out_hbm, idx_v, rows_v, sem):
        wid = lax.axis_index("s") * _NC + lax.axis_index("c")
        pltpu.sync_copy(idx_hbm.at[wid], idx_v)

        def body(c, carry):
            handles = [
                pltpu.async_copy(
                    table_hbm.at[idx_v.at[c * _G + j]],
                    rows_v.at[pl.ds(j * _GRP, _GRP)],
                    sem,
                )
                for j in range(_G)
            ]
            for h in handles:
                h.wait()
            pltpu.sync_copy(rows_v, out_hbm.at[wid, pl.ds(c * _CHUNK, _CHUNK)])
            return carry

        lax.fori_loop(0, n_chunks, body, 0)

    return lookup


def kernel(token_ids, weight):
    b = token_ids.size
    b_per_w = b // _NW
    idx = token_ids.reshape(_NW, b_per_w // _GRP, _GRP).astype(jnp.int32)
    out = _make_lookup(b_per_w)(weight, idx)
    return out.reshape(*token_ids.shape, _D)


# speedup vs baseline: 1.8749x; 1.0184x over previous
import functools

import jax
import jax.numpy as jnp
from jax import lax
from jax.experimental import pallas as pl
from jax.experimental.pallas import tpu as pltpu
from jax.experimental.pallas import tpu_sc as plsc

_NC = 2
_NS = 16
_NW = _NC * _NS
_D = 64
_GRP = 128
_G = 5
_CHUNK = _G * _GRP


@functools.cache
def _make_lookup(b_per_w: int):
    n_grp = b_per_w // _GRP
    n_chunks = b_per_w // _CHUNK
    assert n_chunks % 2 == 0
    mesh = plsc.VectorSubcoreMesh(core_axis_name="c", subcore_axis_name="s")

    @functools.partial(
        pl.kernel,
        mesh=mesh,
        compiler_params=pltpu.CompilerParams(use_tc_tiling_on_sc=False),
        out_type=jax.ShapeDtypeStruct((_NW, b_per_w, _D), jnp.float32),
        scratch_types=[
            pltpu.VMEM((n_grp, _GRP), jnp.int32),
            pltpu.VMEM((2, _CHUNK, _D), jnp.float32),
            pltpu.SemaphoreType.DMA,
            pltpu.SemaphoreType.DMA,
            pltpu.SemaphoreType.DMA,
            pltpu.SemaphoreType.DMA,
        ],
    )
    def lookup(table_hbm, idx_hbm, out_hbm, idx_v, rows_v, sg0, sg1, so0, so1):
        wid = lax.axis_index("s") * _NC + lax.axis_index("c")
        pltpu.sync_copy(idx_hbm.at[wid], idx_v)
        sg = (sg0, sg1)
        so = (so0, so1)

        def fire_g(c, b):
            for j in range(_G):
                pltpu.async_copy(
                    table_hbm.at[idx_v.at[c * _G + j]],
                    rows_v.at[b].at[pl.ds(j * _GRP, _GRP)],
                    sg[b],
                )

        def wait_g(b):
            pltpu.make_async_copy(
                out_hbm.at[wid, pl.ds(0, _CHUNK)], rows_v.at[b], sg[b]
            ).wait()

        def fire_o(c, b):
            pltpu.async_copy(
                rows_v.at[b], out_hbm.at[wid, pl.ds(c * _CHUNK, _CHUNK)], so[b]
            )

        def wait_o(b):
            pltpu.make_async_copy(
                rows_v.at[b], out_hbm.at[wid, pl.ds(0, _CHUNK)], so[b]
            ).wait()

        def body(i, carry):
            c0 = 2 * i

            @pl.when(i > 0)
            def _():
                wait_o(0)

            fire_g(c0, 0)

            @pl.when(i > 0)
            def _():
                wait_g(1)
                fire_o(c0 - 1, 1)
                wait_o(1)

            fire_g(c0 + 1, 1)
            wait_g(0)
            fire_o(c0, 0)
            return carry

        lax.fori_loop(0, n_chunks // 2, body, 0)
        wait_g(1)
        fire_o(n_chunks - 1, 1)
        wait_o(0)
        wait_o(1)

    return lookup


def kernel(token_ids, weight):
    b = token_ids.size
    b_per_w = b // _NW
    idx = token_ids.reshape(_NW, b_per_w // _GRP, _GRP).astype(jnp.int32)
    out = _make_lookup(b_per_w)(weight, idx)
    return out.reshape(*token_ids.shape, _D)
